# baseline (speedup 1.0000x reference)
import functools
import jax
import jax.numpy as jnp
from jax import lax
from jax.experimental import pallas as pl
from jax.experimental.pallas import tpu as pltpu
from jax.experimental.pallas import tpu_sc as plsc

_THRESHOLD = 0.5
_NB_AVERAGE = 9
_OFFSET = _NB_AVERAGE // 2

_L = 16
_NSLICE = 23
_G = 4
_CHUNK = _G * _L
_NW = 32


def _sc_call(x_flat, fbins, rows, n_bins):
    rows_w = rows // _NW
    nchunks = rows_w // _CHUNK
    chunk_words = _CHUNK * n_bins
    mesh = plsc.VectorSubcoreMesh(core_axis_name="c", subcore_axis_name="s")

    @functools.partial(
        pl.kernel,
        mesh=mesh,
        out_type=[
            jax.ShapeDtypeStruct((rows,), jnp.float32),
            jax.ShapeDtypeStruct((rows,), jnp.float32),
        ],
        scratch_types=[
            pltpu.VMEM((2 * chunk_words + _L,), jnp.float32),
            pltpu.VMEM((384,), jnp.float32),
            pltpu.VMEM((_CHUNK,), jnp.float32),
            pltpu.VMEM((_CHUNK,), jnp.float32),
            pltpu.SemaphoreType.DMA,
            pltpu.SemaphoreType.DMA,
            pltpu.SemaphoreType.DMA,
        ],
        compiler_params=pltpu.CompilerParams(needs_layout_passes=False),
    )
    def k(x_hbm, fb_hbm, f_hbm, c_hbm, buf, fbv, fres, cres,
          sem_a, sem_b, sem_out):
        wid = lax.axis_index("s") * 2 + lax.axis_index("c")
        base_row = wid * rows_w
        pltpu.sync_copy(fb_hbm, fbv.at[pl.ds(0, 360)])
        lanes = lax.iota(jnp.int32, _L)
        zeros_f = jnp.zeros((_L,), jnp.float32)
        gmask = lanes < _NB_AVERAGE
        sems = (sem_a, sem_b)

        def start_fetch(ci, slot):
            row0 = base_row + ci * _CHUNK
            return pltpu.async_copy(
                x_hbm.at[pl.ds(row0 * n_bins, chunk_words)],
                buf.at[pl.ds(slot * chunk_words, chunk_words)],
                sems[slot],
            )

        def bfly(v, op):
            for s in (8, 4, 2, 1):
                perm = jnp.bitwise_xor(lanes, s)
                v = op(v, v.at[perm].get(mode="promise_in_bounds"))
            return v

        start_fetch(0, 0)

        def process_chunk(ci, slot):
            boff = slot * chunk_words
            row0c = base_row + ci * _CHUNK
            pltpu.make_async_copy(
                x_hbm.at[pl.ds(row0c * n_bins, chunk_words)],
                buf.at[pl.ds(boff, chunk_words)],
                sems[slot],
            ).wait()

            @pl.when(ci + 1 < nchunks)
            def _():
                start_fetch(ci + 1, 1 - slot)

            def group_body(g, carry2):
                goff = boff + g * (_L * n_bins)
                psacc = zeros_f
                wsacc = zeros_f
                macc = zeros_f
                for rr in range(_L):
                    roff = goff + rr * n_bins
                    vs = []
                    for kk in range(_NSLICE):
                        v = buf[pl.ds(roff + kk * _L, _L)]
                        if kk == _NSLICE - 1:
                            v = jnp.where(lanes < (n_bins - kk * _L), v,
                                          -jnp.inf)
                        vs.append(v)
                    tree = vs
                    while len(tree) > 1:
                        nxt = [jnp.maximum(tree[i], tree[i + 1])
                               for i in range(0, len(tree) - 1, 2)]
                        if len(tree) % 2:
                            nxt.append(tree[-1])
                        tree = nxt
                    mmax = bfly(tree[0], jnp.maximum)
                    cands = [jnp.where(vs[kk] == mmax, kk * _L + lanes, n_bins)
                             for kk in range(_NSLICE)]
                    while len(cands) > 1:
                        nxt = [jnp.minimum(cands[i], cands[i + 1])
                               for i in range(0, len(cands) - 1, 2)]
                        if len(cands) % 2:
                            nxt.append(cands[-1])
                        cands = nxt
                    center = bfly(cands[0], jnp.minimum)
                    start = jnp.clip(center - _OFFSET, 0, n_bins - _NB_AVERAGE)
                    gidx = start + lanes
                    w = plsc.load_gather(buf, [roff + gidx])
                    cc = plsc.load_gather(fbv, [gidx])
                    w = jnp.where(gmask, w, 0.0)
                    cc = jnp.where(gmask, cc, 0.0)
                    wsum = bfly(w, jnp.add)
                    psum = bfly(w * cc, jnp.add)
                    lane_rr = lanes == rr
                    psacc = jnp.where(lane_rr, psum, psacc)
                    wsacc = jnp.where(lane_rr, wsum, wsacc)
                    macc = jnp.where(lane_rr, mmax, macc)
                fv = psacc / wsacc
                voiced = macc > _THRESHOLD
                fres[pl.ds(g * _L, _L)] = jnp.where(voiced, fv, 0.0)
                cres[pl.ds(g * _L, _L)] = jnp.where(voiced, macc, 1.0 - macc)
                return carry2

            lax.fori_loop(0, _G, group_body, 0, unroll=False)
            cp_f = pltpu.async_copy(fres, f_hbm.at[pl.ds(row0c, _CHUNK)],
                                    sem_out)
            cp_c = pltpu.async_copy(cres, c_hbm.at[pl.ds(row0c, _CHUNK)],
                                    sem_out)
            cp_f.wait()
            cp_c.wait()

        def chunk_pair_body(cp, carry):
            process_chunk(2 * cp, 0)
            process_chunk(2 * cp + 1, 1)
            return carry

        lax.fori_loop(0, nchunks // 2, chunk_pair_body, 0, unroll=False)

    return k(x_flat, fbins)


def kernel(inputs, fbins):
    b, t, n_bins = inputs.shape
    rows = b * t
    x_flat = inputs.reshape(rows * n_bins)
    f, c = _sc_call(x_flat, fbins, rows, n_bins)
    return jnp.stack([f.reshape(b, t), c.reshape(b, t)], axis=2)


# speedup vs baseline: 1.2097x; 1.2097x over previous
import functools
import jax
import jax.numpy as jnp
from jax import lax
from jax.experimental import pallas as pl
from jax.experimental.pallas import tpu as pltpu
from jax.experimental.pallas import tpu_sc as plsc

_THRESHOLD = 0.5
_NB_AVERAGE = 9
_OFFSET = _NB_AVERAGE // 2

_L = 16
_NSLICE = 23
_G = 4
_CHUNK = _G * _L
_NW = 32

_TC_BLK = 512
_SC_ROWS = 49152


def _tc_body(x_ref, fb_ref, f_ref, c_ref):
    x = x_ref[...]
    fb = fb_ref[0]
    n_bins = x.shape[-1]
    start_max = n_bins - _NB_AVERAGE

    m = jnp.max(x, axis=-1, keepdims=True)
    iota = jax.lax.broadcasted_iota(jnp.int32, x.shape, 1)
    center = jnp.min(jnp.where(x == m, iota, n_bins), axis=-1, keepdims=True)
    start = jnp.clip(center - _OFFSET, 0, start_max)

    off = (iota - start).astype(jnp.uint32)
    w = jnp.where(off < _NB_AVERAGE, x, 0.0)
    wsum = jnp.sum(w, axis=-1, keepdims=True)
    psum = jnp.sum(w * fb[None, :], axis=-1, keepdims=True)

    f = psum / wsum
    voiced = m > _THRESHOLD
    f_ref[...] = jnp.where(voiced, f, 0.0)[:, 0]
    c_ref[...] = jnp.where(voiced, m, 1.0 - m)[:, 0]


def _tc_call(x2d, fb2d, rows_a, n_bins):
    grid = (rows_a // _TC_BLK,)
    return pl.pallas_call(
        _tc_body,
        grid=grid,
        in_specs=[
            pl.BlockSpec((_TC_BLK, n_bins), lambda i: (i, 0)),
            pl.BlockSpec((1, n_bins), lambda i: (0, 0)),
        ],
        out_specs=[
            pl.BlockSpec((_TC_BLK,), lambda i: (i,)),
            pl.BlockSpec((_TC_BLK,), lambda i: (i,)),
        ],
        out_shape=[
            jax.ShapeDtypeStruct((rows_a,), jnp.float32),
            jax.ShapeDtypeStruct((rows_a,), jnp.float32),
        ],
    )(x2d, fb2d)


def _sc_call(x_flat, fbins, rows_b, n_bins):
    rows_w = rows_b // _NW
    nchunks = rows_w // _CHUNK
    chunk_words = _CHUNK * n_bins
    mesh = plsc.VectorSubcoreMesh(core_axis_name="c", subcore_axis_name="s")

    @functools.partial(
        pl.kernel,
        mesh=mesh,
        out_type=[
            jax.ShapeDtypeStruct((rows_b,), jnp.float32),
            jax.ShapeDtypeStruct((rows_b,), jnp.float32),
        ],
        scratch_types=[
            pltpu.VMEM((2 * chunk_words + _L,), jnp.float32),
            pltpu.VMEM((384,), jnp.float32),
            pltpu.VMEM((_CHUNK,), jnp.float32),
            pltpu.VMEM((_CHUNK,), jnp.float32),
            pltpu.SemaphoreType.DMA,
            pltpu.SemaphoreType.DMA,
            pltpu.SemaphoreType.DMA,
        ],
        compiler_params=pltpu.CompilerParams(needs_layout_passes=False),
    )
    def k(x_hbm, fb_hbm, f_hbm, c_hbm, buf, fbv, fres, cres,
          sem_a, sem_b, sem_out):
        wid = lax.axis_index("s") * 2 + lax.axis_index("c")
        base_row = wid * rows_w
        pltpu.sync_copy(fb_hbm, fbv.at[pl.ds(0, 360)])
        lanes = lax.iota(jnp.int32, _L)
        zeros_f = jnp.zeros((_L,), jnp.float32)
        gmask = lanes < _NB_AVERAGE
        sems = (sem_a, sem_b)

        def start_fetch(ci, slot):
            row0 = base_row + ci * _CHUNK
            return pltpu.async_copy(
                x_hbm.at[pl.ds(row0 * n_bins, chunk_words)],
                buf.at[pl.ds(slot * chunk_words, chunk_words)],
                sems[slot],
            )

        def bfly(v, op):
            for s in (8, 4, 2, 1):
                perm = jnp.bitwise_xor(lanes, s)
                v = op(v, v.at[perm].get(mode="promise_in_bounds"))
            return v

        start_fetch(0, 0)

        def process_chunk(ci, slot):
            boff = slot * chunk_words
            row0c = base_row + ci * _CHUNK
            pltpu.make_async_copy(
                x_hbm.at[pl.ds(row0c * n_bins, chunk_words)],
                buf.at[pl.ds(boff, chunk_words)],
                sems[slot],
            ).wait()

            @pl.when(ci + 1 < nchunks)
            def _():
                start_fetch(ci + 1, 1 - slot)

            def group_body(g, carry2):
                goff = boff + g * (_L * n_bins)
                psacc = zeros_f
                wsacc = zeros_f
                macc = zeros_f
                for rr in range(_L):
                    roff = goff + rr * n_bins
                    m = jnp.full((_L,), -jnp.inf, dtype=jnp.float32)
                    bidx = jnp.zeros((_L,), jnp.int32)
                    for kk in range(_NSLICE):
                        v = buf[pl.ds(roff + kk * _L, _L)]
                        if kk == _NSLICE - 1:
                            v = jnp.where(lanes < (n_bins - kk * _L), v,
                                          -jnp.inf)
                        upd = v > m
                        m = jnp.where(upd, v, m)
                        bidx = jnp.where(upd, kk * _L + lanes, bidx)
                    mmax = bfly(m, jnp.maximum)
                    cand = jnp.where(m == mmax, bidx, n_bins)
                    center = bfly(cand, jnp.minimum)
                    start = jnp.clip(center - _OFFSET, 0, n_bins - _NB_AVERAGE)
                    gidx = start + lanes
                    w = plsc.load_gather(buf, [roff + gidx])
                    cc = plsc.load_gather(fbv, [gidx])
                    w = jnp.where(gmask, w, 0.0)
                    cc = jnp.where(gmask, cc, 0.0)
                    wsum = bfly(w, jnp.add)
                    psum = bfly(w * cc, jnp.add)
                    lane_rr = lanes == rr
                    psacc = jnp.where(lane_rr, psum, psacc)
                    wsacc = jnp.where(lane_rr, wsum, wsacc)
                    macc = jnp.where(lane_rr, mmax, macc)
                fv = psacc / wsacc
                voiced = macc > _THRESHOLD
                fres[pl.ds(g * _L, _L)] = jnp.where(voiced, fv, 0.0)
                cres[pl.ds(g * _L, _L)] = jnp.where(voiced, macc, 1.0 - macc)
                return carry2

            lax.fori_loop(0, _G, group_body, 0, unroll=False)
            cp_f = pltpu.async_copy(fres, f_hbm.at[pl.ds(row0c, _CHUNK)],
                                    sem_out)
            cp_c = pltpu.async_copy(cres, c_hbm.at[pl.ds(row0c, _CHUNK)],
                                    sem_out)
            cp_f.wait()
            cp_c.wait()

        def chunk_pair_body(cp, carry):
            process_chunk(2 * cp, 0)
            process_chunk(2 * cp + 1, 1)
            return carry

        lax.fori_loop(0, nchunks // 2, chunk_pair_body, 0, unroll=False)

    return k(x_flat, fbins)


def kernel(inputs, fbins):
    b, t, n_bins = inputs.shape
    rows = b * t
    rows_a = rows - _SC_ROWS
    x2d = inputs.reshape(rows, n_bins)
    fa, ca = _tc_call(x2d[:rows_a], fbins.reshape(1, n_bins), rows_a, n_bins)
    x_flat_b = x2d[rows_a:].reshape(_SC_ROWS * n_bins)
    fb_, cb = _sc_call(x_flat_b, fbins, _SC_ROWS, n_bins)
    f = jnp.concatenate([fa, fb_]).reshape(b, t)
    c = jnp.concatenate([ca, cb]).reshape(b, t)
    return jnp.stack([f, c], axis=2)


# speedup vs baseline: 1.3283x; 1.0981x over previous
import functools
import jax
import jax.numpy as jnp
from jax import lax
from jax.experimental import pallas as pl
from jax.experimental.pallas import tpu as pltpu
from jax.experimental.pallas import tpu_sc as plsc

_THRESHOLD = 0.5
_NB_AVERAGE = 9
_OFFSET = _NB_AVERAGE // 2

_L = 16
_NSLICE = 23
_G = 4
_CHUNK = _G * _L
_NW = 32

_TC_BLK = 512
_SC_ROWS = 131072


def _tc_body(x_ref, fb_ref, f_ref, c_ref):
    x = x_ref[...]
    fb = fb_ref[0]
    n_bins = x.shape[-1]
    start_max = n_bins - _NB_AVERAGE

    m = jnp.max(x, axis=-1, keepdims=True)
    iota = jax.lax.broadcasted_iota(jnp.int32, x.shape, 1)
    center = jnp.min(jnp.where(x == m, iota, n_bins), axis=-1, keepdims=True)
    start = jnp.clip(center - _OFFSET, 0, start_max)

    off = (iota - start).astype(jnp.uint32)
    w = jnp.where(off < _NB_AVERAGE, x, 0.0)
    wsum = jnp.sum(w, axis=-1, keepdims=True)
    psum = jnp.sum(w * fb[None, :], axis=-1, keepdims=True)

    f = psum / wsum
    voiced = m > _THRESHOLD
    f_ref[...] = jnp.where(voiced, f, 0.0)[:, 0]
    c_ref[...] = jnp.where(voiced, m, 1.0 - m)[:, 0]


def _tc_call(x2d, fb2d, rows_a, n_bins):
    grid = (rows_a // _TC_BLK,)
    return pl.pallas_call(
        _tc_body,
        grid=grid,
        in_specs=[
            pl.BlockSpec((_TC_BLK, n_bins), lambda i: (i, 0)),
            pl.BlockSpec((1, n_bins), lambda i: (0, 0)),
        ],
        out_specs=[
            pl.BlockSpec((_TC_BLK,), lambda i: (i,)),
            pl.BlockSpec((_TC_BLK,), lambda i: (i,)),
        ],
        out_shape=[
            jax.ShapeDtypeStruct((rows_a,), jnp.float32),
            jax.ShapeDtypeStruct((rows_a,), jnp.float32),
        ],
    )(x2d, fb2d)


def _sc_call(x_flat, fbins, rows_b, n_bins):
    rows_w = rows_b // _NW
    nchunks = rows_w // _CHUNK
    chunk_words = _CHUNK * n_bins
    mesh = plsc.VectorSubcoreMesh(core_axis_name="c", subcore_axis_name="s")

    @functools.partial(
        pl.kernel,
        mesh=mesh,
        out_type=[
            jax.ShapeDtypeStruct((rows_b,), jnp.float32),
            jax.ShapeDtypeStruct((rows_b,), jnp.float32),
        ],
        scratch_types=[
            pltpu.VMEM((2 * chunk_words + _L,), jnp.float32),
            pltpu.VMEM((384,), jnp.float32),
            pltpu.VMEM((_CHUNK,), jnp.float32),
            pltpu.VMEM((_CHUNK,), jnp.float32),
            pltpu.SemaphoreType.DMA,
            pltpu.SemaphoreType.DMA,
            pltpu.SemaphoreType.DMA,
        ],
        compiler_params=pltpu.CompilerParams(needs_layout_passes=False),
    )
    def k(x_hbm, fb_hbm, f_hbm, c_hbm, buf, fbv, fres, cres,
          sem_a, sem_b, sem_out):
        wid = lax.axis_index("s") * 2 + lax.axis_index("c")
        base_row = wid * rows_w
        pltpu.sync_copy(fb_hbm, fbv.at[pl.ds(0, 360)])
        lanes = lax.iota(jnp.int32, _L)
        zeros_f = jnp.zeros((_L,), jnp.float32)
        gmask = lanes < _NB_AVERAGE
        sems = (sem_a, sem_b)

        def start_fetch(ci, slot):
            row0 = base_row + ci * _CHUNK
            return pltpu.async_copy(
                x_hbm.at[pl.ds(row0 * n_bins, chunk_words)],
                buf.at[pl.ds(slot * chunk_words, chunk_words)],
                sems[slot],
            )

        def bfly(v, op):
            for s in (8, 4, 2, 1):
                perm = jnp.bitwise_xor(lanes, s)
                v = op(v, v.at[perm].get(mode="promise_in_bounds"))
            return v

        start_fetch(0, 0)

        def process_chunk(ci, slot):
            boff = slot * chunk_words
            row0c = base_row + ci * _CHUNK
            pltpu.make_async_copy(
                x_hbm.at[pl.ds(row0c * n_bins, chunk_words)],
                buf.at[pl.ds(boff, chunk_words)],
                sems[slot],
            ).wait()

            @pl.when(ci + 1 < nchunks)
            def _():
                start_fetch(ci + 1, 1 - slot)

            def group_body(g, carry2):
                goff = boff + g * (_L * n_bins)
                psacc = zeros_f
                wsacc = zeros_f
                macc = zeros_f
                for rr in range(_L):
                    roff = goff + rr * n_bins
                    m = jnp.full((_L,), -jnp.inf, dtype=jnp.float32)
                    bidx = jnp.zeros((_L,), jnp.int32)
                    for kk in range(_NSLICE):
                        v = buf[pl.ds(roff + kk * _L, _L)]
                        if kk == _NSLICE - 1:
                            v = jnp.where(lanes < (n_bins - kk * _L), v,
                                          -jnp.inf)
                        upd = v > m
                        m = jnp.where(upd, v, m)
                        bidx = jnp.where(upd, kk * _L + lanes, bidx)
                    mmax = bfly(m, jnp.maximum)
                    cand = jnp.where(m == mmax, bidx, n_bins)
                    center = bfly(cand, jnp.minimum)
                    start = jnp.clip(center - _OFFSET, 0, n_bins - _NB_AVERAGE)
                    gidx = start + lanes
                    w = plsc.load_gather(buf, [roff + gidx])
                    cc = plsc.load_gather(fbv, [gidx])
                    w = jnp.where(gmask, w, 0.0)
                    cc = jnp.where(gmask, cc, 0.0)
                    wsum = bfly(w, jnp.add)
                    psum = bfly(w * cc, jnp.add)
                    lane_rr = lanes == rr
                    psacc = jnp.where(lane_rr, psum, psacc)
                    wsacc = jnp.where(lane_rr, wsum, wsacc)
                    macc = jnp.where(lane_rr, mmax, macc)
                fv = psacc / wsacc
                voiced = macc > _THRESHOLD
                fres[pl.ds(g * _L, _L)] = jnp.where(voiced, fv, 0.0)
                cres[pl.ds(g * _L, _L)] = jnp.where(voiced, macc, 1.0 - macc)
                return carry2

            lax.fori_loop(0, _G, group_body, 0, unroll=False)
            cp_f = pltpu.async_copy(fres, f_hbm.at[pl.ds(row0c, _CHUNK)],
                                    sem_out)
            cp_c = pltpu.async_copy(cres, c_hbm.at[pl.ds(row0c, _CHUNK)],
                                    sem_out)
            cp_f.wait()
            cp_c.wait()

        def chunk_pair_body(cp, carry):
            process_chunk(2 * cp, 0)
            process_chunk(2 * cp + 1, 1)
            return carry

        lax.fori_loop(0, nchunks // 2, chunk_pair_body, 0, unroll=False)

    return k(x_flat, fbins)


def kernel(inputs, fbins):
    b, t, n_bins = inputs.shape
    rows = b * t
    x_flat = inputs.reshape(rows * n_bins)
    f, c = _sc_call(x_flat, fbins, rows, n_bins)
    return jnp.stack([f.reshape(b, t), c.reshape(b, t)], axis=2)


# speedup vs baseline: 1.4676x; 1.1049x over previous
import functools
import jax
import jax.numpy as jnp
from jax import lax
from jax.experimental import pallas as pl
from jax.experimental.pallas import tpu as pltpu
from jax.experimental.pallas import tpu_sc as plsc

_THRESHOLD = 0.5
_NB_AVERAGE = 9
_OFFSET = _NB_AVERAGE // 2

_L = 16
_NSLICE = 23
_G = 4
_CHUNK = _G * _L
_NW = 32

_TC_BLK = 512
_SC_ROWS = 131072


def _tc_body(x_ref, fb_ref, f_ref, c_ref):
    x = x_ref[...]
    fb = fb_ref[0]
    n_bins = x.shape[-1]
    start_max = n_bins - _NB_AVERAGE

    m = jnp.max(x, axis=-1, keepdims=True)
    iota = jax.lax.broadcasted_iota(jnp.int32, x.shape, 1)
    center = jnp.min(jnp.where(x == m, iota, n_bins), axis=-1, keepdims=True)
    start = jnp.clip(center - _OFFSET, 0, start_max)

    off = (iota - start).astype(jnp.uint32)
    w = jnp.where(off < _NB_AVERAGE, x, 0.0)
    wsum = jnp.sum(w, axis=-1, keepdims=True)
    psum = jnp.sum(w * fb[None, :], axis=-1, keepdims=True)

    f = psum / wsum
    voiced = m > _THRESHOLD
    f_ref[...] = jnp.where(voiced, f, 0.0)[:, 0]
    c_ref[...] = jnp.where(voiced, m, 1.0 - m)[:, 0]


def _tc_call(x2d, fb2d, rows_a, n_bins):
    grid = (rows_a // _TC_BLK,)
    return pl.pallas_call(
        _tc_body,
        grid=grid,
        in_specs=[
            pl.BlockSpec((_TC_BLK, n_bins), lambda i: (i, 0)),
            pl.BlockSpec((1, n_bins), lambda i: (0, 0)),
        ],
        out_specs=[
            pl.BlockSpec((_TC_BLK,), lambda i: (i,)),
            pl.BlockSpec((_TC_BLK,), lambda i: (i,)),
        ],
        out_shape=[
            jax.ShapeDtypeStruct((rows_a,), jnp.float32),
            jax.ShapeDtypeStruct((rows_a,), jnp.float32),
        ],
    )(x2d, fb2d)


def _sc_call(x_flat, fbins, rows_b, n_bins):
    rows_w = rows_b // _NW
    nchunks = rows_w // _CHUNK
    chunk_words = _CHUNK * n_bins
    mesh = plsc.VectorSubcoreMesh(core_axis_name="c", subcore_axis_name="s")

    @functools.partial(
        pl.kernel,
        mesh=mesh,
        out_type=[
            jax.ShapeDtypeStruct((rows_b,), jnp.float32),
            jax.ShapeDtypeStruct((rows_b,), jnp.float32),
        ],
        scratch_types=[
            pltpu.VMEM((2 * chunk_words + _L,), jnp.float32),
            pltpu.VMEM((384,), jnp.float32),
            pltpu.VMEM((_CHUNK,), jnp.float32),
            pltpu.VMEM((_CHUNK,), jnp.float32),
            pltpu.SemaphoreType.DMA,
            pltpu.SemaphoreType.DMA,
            pltpu.SemaphoreType.DMA,
        ],
        compiler_params=pltpu.CompilerParams(needs_layout_passes=False),
    )
    def k(x_hbm, fb_hbm, f_hbm, c_hbm, buf, fbv, fres, cres,
          sem_a, sem_b, sem_out):
        wid = lax.axis_index("s") * 2 + lax.axis_index("c")
        base_row = wid * rows_w
        pltpu.sync_copy(fb_hbm, fbv.at[pl.ds(0, 360)])
        lanes = lax.iota(jnp.int32, _L)
        zeros_f = jnp.zeros((_L,), jnp.float32)
        gmask = lanes < _NB_AVERAGE
        sems = (sem_a, sem_b)

        def start_fetch(ci, slot):
            row0 = base_row + ci * _CHUNK
            return pltpu.async_copy(
                x_hbm.at[pl.ds(row0 * n_bins, chunk_words)],
                buf.at[pl.ds(slot * chunk_words, chunk_words)],
                sems[slot],
            )

        def bfly(v, op):
            for s in (8, 4, 2, 1):
                perm = jnp.bitwise_xor(lanes, s)
                v = op(v, v.at[perm].get(mode="promise_in_bounds"))
            return v

        start_fetch(0, 0)

        def process_chunk(ci, slot):
            boff = slot * chunk_words
            row0c = base_row + ci * _CHUNK
            pltpu.make_async_copy(
                x_hbm.at[pl.ds(row0c * n_bins, chunk_words)],
                buf.at[pl.ds(boff, chunk_words)],
                sems[slot],
            ).wait()

            @pl.when(ci + 1 < nchunks)
            def _():
                start_fetch(ci + 1, 1 - slot)

            def group_body(g, carry2):
                goff = boff + g * (_L * n_bins)
                psacc = zeros_f
                wsacc = zeros_f
                macc = zeros_f
                for rr0 in range(0, _L, 4):
                    roffs = [goff + (rr0 + j) * n_bins for j in range(4)]
                    ms = [jnp.full((_L,), -jnp.inf, dtype=jnp.float32)
                          for _ in range(4)]
                    bidxs = [jnp.zeros((_L,), jnp.int32) for _ in range(4)]
                    for kk in range(_NSLICE):
                        for j in range(4):
                            v = buf[pl.ds(roffs[j] + kk * _L, _L)]
                            if kk == _NSLICE - 1:
                                v = jnp.where(lanes < (n_bins - kk * _L), v,
                                              -jnp.inf)
                            upd = v > ms[j]
                            ms[j] = jnp.where(upd, v, ms[j])
                            bidxs[j] = jnp.where(upd, kk * _L + lanes,
                                                 bidxs[j])
                    for j in range(4):
                        rr = rr0 + j
                        mmax = bfly(ms[j], jnp.maximum)
                        cand = jnp.where(ms[j] == mmax, bidxs[j], n_bins)
                        center = bfly(cand, jnp.minimum)
                        start = jnp.clip(center - _OFFSET,
                                         0, n_bins - _NB_AVERAGE)
                        gidx = start + lanes
                        w = plsc.load_gather(buf, [roffs[j] + gidx])
                        cc = plsc.load_gather(fbv, [gidx])
                        w = jnp.where(gmask, w, 0.0)
                        cc = jnp.where(gmask, cc, 0.0)
                        wsum = bfly(w, jnp.add)
                        psum = bfly(w * cc, jnp.add)
                        lane_rr = lanes == rr
                        psacc = jnp.where(lane_rr, psum, psacc)
                        wsacc = jnp.where(lane_rr, wsum, wsacc)
                        macc = jnp.where(lane_rr, mmax, macc)
                fv = psacc / wsacc
                voiced = macc > _THRESHOLD
                fres[pl.ds(g * _L, _L)] = jnp.where(voiced, fv, 0.0)
                cres[pl.ds(g * _L, _L)] = jnp.where(voiced, macc, 1.0 - macc)
                return carry2

            lax.fori_loop(0, _G, group_body, 0, unroll=False)
            cp_f = pltpu.async_copy(fres, f_hbm.at[pl.ds(row0c, _CHUNK)],
                                    sem_out)
            cp_c = pltpu.async_copy(cres, c_hbm.at[pl.ds(row0c, _CHUNK)],
                                    sem_out)
            cp_f.wait()
            cp_c.wait()

        def chunk_pair_body(cp, carry):
            process_chunk(2 * cp, 0)
            process_chunk(2 * cp + 1, 1)
            return carry

        lax.fori_loop(0, nchunks // 2, chunk_pair_body, 0, unroll=False)

    return k(x_flat, fbins)


def kernel(inputs, fbins):
    b, t, n_bins = inputs.shape
    rows = b * t
    x_flat = inputs.reshape(rows * n_bins)
    f, c = _sc_call(x_flat, fbins, rows, n_bins)
    return jnp.stack([f.reshape(b, t), c.reshape(b, t)], axis=2)


# speedup vs baseline: 1.4726x; 1.0034x over previous
import functools
import jax
import jax.numpy as jnp
from jax import lax
from jax.experimental import pallas as pl
from jax.experimental.pallas import tpu as pltpu
from jax.experimental.pallas import tpu_sc as plsc

_THRESHOLD = 0.5
_NB_AVERAGE = 9
_OFFSET = _NB_AVERAGE // 2

_L = 16
_NSLICE = 23
_G = 8
_CHUNK = _G * _L
_NW = 32

_TC_BLK = 512
_SC_ROWS = 131072


def _tc_body(x_ref, fb_ref, f_ref, c_ref):
    x = x_ref[...]
    fb = fb_ref[0]
    n_bins = x.shape[-1]
    start_max = n_bins - _NB_AVERAGE

    m = jnp.max(x, axis=-1, keepdims=True)
    iota = jax.lax.broadcasted_iota(jnp.int32, x.shape, 1)
    center = jnp.min(jnp.where(x == m, iota, n_bins), axis=-1, keepdims=True)
    start = jnp.clip(center - _OFFSET, 0, start_max)

    off = (iota - start).astype(jnp.uint32)
    w = jnp.where(off < _NB_AVERAGE, x, 0.0)
    wsum = jnp.sum(w, axis=-1, keepdims=True)
    psum = jnp.sum(w * fb[None, :], axis=-1, keepdims=True)

    f = psum / wsum
    voiced = m > _THRESHOLD
    f_ref[...] = jnp.where(voiced, f, 0.0)[:, 0]
    c_ref[...] = jnp.where(voiced, m, 1.0 - m)[:, 0]


def _tc_call(x2d, fb2d, rows_a, n_bins):
    grid = (rows_a // _TC_BLK,)
    return pl.pallas_call(
        _tc_body,
        grid=grid,
        in_specs=[
            pl.BlockSpec((_TC_BLK, n_bins), lambda i: (i, 0)),
            pl.BlockSpec((1, n_bins), lambda i: (0, 0)),
        ],
        out_specs=[
            pl.BlockSpec((_TC_BLK,), lambda i: (i,)),
            pl.BlockSpec((_TC_BLK,), lambda i: (i,)),
        ],
        out_shape=[
            jax.ShapeDtypeStruct((rows_a,), jnp.float32),
            jax.ShapeDtypeStruct((rows_a,), jnp.float32),
        ],
    )(x2d, fb2d)


def _sc_call(x_flat, fbins, rows_b, n_bins):
    rows_w = rows_b // _NW
    nchunks = rows_w // _CHUNK
    chunk_words = _CHUNK * n_bins
    mesh = plsc.VectorSubcoreMesh(core_axis_name="c", subcore_axis_name="s")

    @functools.partial(
        pl.kernel,
        mesh=mesh,
        out_type=[
            jax.ShapeDtypeStruct((rows_b,), jnp.float32),
            jax.ShapeDtypeStruct((rows_b,), jnp.float32),
        ],
        scratch_types=[
            pltpu.VMEM((2 * chunk_words + _L,), jnp.float32),
            pltpu.VMEM((384,), jnp.float32),
            pltpu.VMEM((_CHUNK,), jnp.float32),
            pltpu.VMEM((_CHUNK,), jnp.float32),
            pltpu.SemaphoreType.DMA,
            pltpu.SemaphoreType.DMA,
            pltpu.SemaphoreType.DMA,
        ],
        compiler_params=pltpu.CompilerParams(needs_layout_passes=False),
    )
    def k(x_hbm, fb_hbm, f_hbm, c_hbm, buf, fbv, fres, cres,
          sem_a, sem_b, sem_out):
        wid = lax.axis_index("s") * 2 + lax.axis_index("c")
        base_row = wid * rows_w
        pltpu.sync_copy(fb_hbm, fbv.at[pl.ds(0, 360)])
        lanes = lax.iota(jnp.int32, _L)
        zeros_f = jnp.zeros((_L,), jnp.float32)
        gmask = lanes < _NB_AVERAGE
        sems = (sem_a, sem_b)

        def start_fetch(ci, slot):
            row0 = base_row + ci * _CHUNK
            return pltpu.async_copy(
                x_hbm.at[pl.ds(row0 * n_bins, chunk_words)],
                buf.at[pl.ds(slot * chunk_words, chunk_words)],
                sems[slot],
            )

        def bfly(v, op):
            for s in (8, 4, 2, 1):
                perm = jnp.bitwise_xor(lanes, s)
                v = op(v, v.at[perm].get(mode="promise_in_bounds"))
            return v

        start_fetch(0, 0)

        def process_chunk(ci, slot):
            boff = slot * chunk_words
            row0c = base_row + ci * _CHUNK
            pltpu.make_async_copy(
                x_hbm.at[pl.ds(row0c * n_bins, chunk_words)],
                buf.at[pl.ds(boff, chunk_words)],
                sems[slot],
            ).wait()

            @pl.when(ci + 1 < nchunks)
            def _():
                start_fetch(ci + 1, 1 - slot)

            def group_body(g, carry2):
                goff = boff + g * (_L * n_bins)
                psacc = zeros_f
                wsacc = zeros_f
                macc = zeros_f
                for rr0 in range(0, _L, 4):
                    roffs = [goff + (rr0 + j) * n_bins for j in range(4)]
                    ms = [jnp.full((_L,), -jnp.inf, dtype=jnp.float32)
                          for _ in range(4)]
                    bidxs = [jnp.zeros((_L,), jnp.int32) for _ in range(4)]
                    for kk in range(_NSLICE):
                        for j in range(4):
                            v = buf[pl.ds(roffs[j] + kk * _L, _L)]
                            if kk == _NSLICE - 1:
                                v = jnp.where(lanes < (n_bins - kk * _L), v,
                                              -jnp.inf)
                            upd = v > ms[j]
                            ms[j] = jnp.where(upd, v, ms[j])
                            bidxs[j] = jnp.where(upd, kk * _L + lanes,
                                                 bidxs[j])
                    for j in range(4):
                        rr = rr0 + j
                        mmax = bfly(ms[j], jnp.maximum)
                        cand = jnp.where(ms[j] == mmax, bidxs[j], n_bins)
                        center = bfly(cand, jnp.minimum)
                        start = jnp.clip(center - _OFFSET,
                                         0, n_bins - _NB_AVERAGE)
                        gidx = start + lanes
                        w = plsc.load_gather(buf, [roffs[j] + gidx])
                        cc = plsc.load_gather(fbv, [gidx])
                        w = jnp.where(gmask, w, 0.0)
                        cc = jnp.where(gmask, cc, 0.0)
                        wsum = bfly(w, jnp.add)
                        psum = bfly(w * cc, jnp.add)
                        lane_rr = lanes == rr
                        psacc = jnp.where(lane_rr, psum, psacc)
                        wsacc = jnp.where(lane_rr, wsum, wsacc)
                        macc = jnp.where(lane_rr, mmax, macc)
                fv = psacc / wsacc
                voiced = macc > _THRESHOLD
                fres[pl.ds(g * _L, _L)] = jnp.where(voiced, fv, 0.0)
                cres[pl.ds(g * _L, _L)] = jnp.where(voiced, macc, 1.0 - macc)
                return carry2

            lax.fori_loop(0, _G, group_body, 0, unroll=False)
            cp_f = pltpu.async_copy(fres, f_hbm.at[pl.ds(row0c, _CHUNK)],
                                    sem_out)
            cp_c = pltpu.async_copy(cres, c_hbm.at[pl.ds(row0c, _CHUNK)],
                                    sem_out)
            cp_f.wait()
            cp_c.wait()

        def chunk_pair_body(cp, carry):
            process_chunk(2 * cp, 0)
            process_chunk(2 * cp + 1, 1)
            return carry

        lax.fori_loop(0, nchunks // 2, chunk_pair_body, 0, unroll=False)

    return k(x_flat, fbins)


def kernel(inputs, fbins):
    b, t, n_bins = inputs.shape
    rows = b * t
    x_flat = inputs.reshape(rows * n_bins)
    f, c = _sc_call(x_flat, fbins, rows, n_bins)
    return jnp.stack([f.reshape(b, t), c.reshape(b, t)], axis=2)
